# baseline (speedup 1.0000x reference)
import functools

import jax
import jax.numpy as jnp
from jax import lax
from jax.experimental import pallas as pl
from jax.experimental.pallas import tpu as pltpu
from jax.experimental.pallas import tpu_sc as plsc

_VOCAB = 64
_IN = 64
_HID = 13
_N = 1024
_SEQ = 16
_E = 16384
_G = 4 * _HID

_HP = lax.Precision.DEFAULT


def _dot(a, b):
    return jnp.dot(a, b, precision=_HP, preferred_element_type=jnp.float32)


def _cell(g, c):
    i = jax.nn.sigmoid(g[:, 0:_HID])
    f = jax.nn.sigmoid(g[:, _HID:2 * _HID])
    gg = jnp.tanh(g[:, 2 * _HID:3 * _HID])
    o = jax.nn.sigmoid(g[:, 3 * _HID:4 * _HID])
    c2 = f * c + i * gg
    return o * jnp.tanh(c2), c2


def _cell_t(g, c):
    i = jax.nn.sigmoid(g[0:32])
    f = jax.nn.sigmoid(g[32:64])
    gg = jnp.tanh(g[64:96])
    o = jax.nn.sigmoid(g[96:128])
    c2 = f * c + i * gg
    return o * jnp.tanh(c2), c2


def _cell_t52(g, c):
    i = jax.nn.sigmoid(g[0:_HID])
    f = jax.nn.sigmoid(g[_HID:2 * _HID])
    gg = jnp.tanh(g[2 * _HID:3 * _HID])
    o = jax.nn.sigmoid(g[3 * _HID:4 * _HID])
    c2 = f * c + i * gg
    return o * jnp.tanh(c2)


def _leaky(v):
    return jnp.where(v >= 0, v, 0.01 * v)


def _sc_counts(edge_index):
    info = plsc.get_sparse_core_info()
    nc, ns = info.num_cores, info.num_subcores
    ntiles = nc * ns
    e_per = _E // ntiles
    chunk = 128
    nchunk = e_per // chunk
    words_per_tile = (_N * _N) // ns
    zb = 2048

    mesh = plsc.VectorSubcoreMesh(core_axis_name="c", subcore_axis_name="s")

    @functools.partial(
        pl.kernel,
        out_type=jax.ShapeDtypeStruct((nc, _N * _N), jnp.float32),
        mesh=mesh,
        scratch_types=[
            pltpu.VMEM((e_per,), jnp.int32),
            pltpu.VMEM((e_per,), jnp.int32),
            pltpu.VMEM((nchunk, chunk), jnp.int32),
            pltpu.VMEM((chunk,), jnp.float32),
            pltpu.VMEM((zb,), jnp.float32),
            pltpu.VMEM_SHARED((_N * _N,), jnp.float32),
        ],
    )
    def counts_kernel(edges, out, src_v, dst_v, idx_v, ones_v, zb_v, acc):
        cid = lax.axis_index("c")
        sid = lax.axis_index("s")
        gid = cid * ns + sid

        zeros16 = jnp.zeros((16,), jnp.float32)
        ones16 = jnp.ones((16,), jnp.float32)
        for k in range(zb // 16):
            zb_v[pl.ds(k * 16, 16)] = zeros16
        for k in range(chunk // 16):
            ones_v[pl.ds(k * 16, 16)] = ones16

        base = gid * e_per
        pltpu.sync_copy(edges.at[0, pl.ds(base, e_per)], src_v)
        pltpu.sync_copy(edges.at[1, pl.ds(base, e_per)], dst_v)
        for j in range(nchunk):
            for k in range(chunk // 16):
                o = j * chunk + k * 16
                s16 = src_v[pl.ds(o, 16)]
                d16 = dst_v[pl.ds(o, 16)]
                idx_v[j, pl.ds(k * 16, 16)] = s16 * _N + d16

        zbase = sid * words_per_tile
        for j in range(words_per_tile // zb):
            pltpu.sync_copy(zb_v, acc.at[pl.ds(zbase + j * zb, zb)])
        plsc.subcore_barrier()

        for j in range(nchunk):
            pltpu.sync_copy(ones_v, acc.at[idx_v.at[j]], add=True)
        plsc.subcore_barrier()

        pltpu.sync_copy(acc.at[pl.ds(zbase, words_per_tile)],
                        out.at[cid, pl.ds(zbase, words_per_tile)])

    return counts_kernel(edge_index)


def _enc_body(xr_ref, xt_ref, embt_ref, wf0_ref, wb0_ref, whh0_ref, b0_ref,
              w1f_ref, w1b_ref, whh1_ref, b1_ref,
              p1w0_ref, p1w1_ref, p1b_ref, p2w0_ref, p2w1_ref, p2b_ref,
              dwih_ref, db_ref, ow_ref, ob_ref,
              tail_ref, sht_ref, sct_ref, hs_ref):
    iota_t = lax.broadcasted_iota(jnp.int32, (_VOCAB, _N), 0)

    def oh_t(t):
        return (xr_ref[t] == iota_t).astype(jnp.float32)

    embt = embt_ref[...]
    ewf = _dot(wf0_ref[...], embt)
    ewb = _dot(wb0_ref[...], embt)
    whh0 = whh0_ref[...]
    b0 = b0_ref[...]
    z = jnp.zeros((2 * _HID + 6, _N), jnp.float32)

    def step0(s, hc):
        h, c = hc
        g = _dot(ewf, oh_t(s)) + _dot(ewb, oh_t(_SEQ - 1 - s))
        g = g + b0 + _dot(whh0, h)
        h, c = _cell_t(g, c)
        hs_ref[s] = h
        return (h, c)

    h0, c0 = lax.fori_loop(0, _SEQ, step0, (z, z))

    whh1 = whh1_ref[...]
    b1 = b1_ref[...]

    def step1(s, hc):
        h, c = hc
        g = _dot(w1f_ref[...], hs_ref[s]) + _dot(w1b_ref[...],
                                                 hs_ref[_SEQ - 1 - s])
        g = g + b1 + _dot(whh1, h)
        return _cell_t(g, c)

    h1, c1 = lax.fori_loop(0, _SEQ, step1, (z, z))

    sht_ref[...] = _dot(p1w0_ref[...], h0) + _dot(p1w1_ref[...], h1) \
        + p1b_ref[...]
    sct_ref[...] = _dot(p2w0_ref[...], c0) + _dot(p2w1_ref[...], c1) \
        + p2b_ref[...]

    vs = lax.broadcasted_iota(jnp.int32, (_VOCAB, 1), 0).astype(jnp.float32)
    zc = jnp.zeros((_VOCAB, _HID), jnp.float32)
    h2s = []
    for d in range(2):
        g = vs * dwih_ref[d] + db_ref[d]
        h2, _ = _cell(g, zc)
        h2s.append(h2)
    table = _dot(jnp.concatenate(h2s, axis=1), ow_ref[...]) + ob_ref[...]
    iota_n = lax.broadcasted_iota(jnp.int32, (_N, _VOCAB), 1)
    for t in range(_SEQ - 1):
        ohn = (xt_ref[t] == iota_n).astype(jnp.float32)
        tail_ref[:, t + 1, :] = _dot(ohn, table)


def _gnn_dec_body(ct_ref, sht_ref, sct_ref,
                  hw1_ref, hb1_ref, hw2_ref, hb2_ref, hfw_ref, hfb_ref,
                  cw1_ref, cb1_ref, cw2_ref, cb2_ref, cfw_ref, cfb_ref,
                  dwf_ref, dhf_ref, dbf_ref, dwb_ref, dhb_ref, dbb_ref,
                  owt_ref, obt_ref, headt_ref):
    ct = ct_ref[0] + ct_ref[1]
    deg = jnp.sum(ct, axis=0, keepdims=True) + 1.0
    dinv = lax.rsqrt(deg)
    d2 = dinv * dinv

    def conv2(hat, hct, wa_ref, ba_ref, wc_ref, bc_ref):
        wa = wa_ref[...]
        hwt = jnp.concatenate([_dot(wa, hat), _dot(wc_ref[...], hct)], axis=0)
        aggt = dinv * _dot(hwt * dinv, ct) + d2 * hwt
        k = wa.shape[0]
        return aggt[:k] + ba_ref[...], aggt[k:] + bc_ref[...]

    a1, c1 = conv2(sht_ref[...], sct_ref[...], hw1_ref, hb1_ref,
                   cw1_ref, cb1_ref)
    a2, c2 = conv2(_leaky(a1), _leaky(c1), hw2_ref, hb2_ref,
                   cw2_ref, cb2_ref)
    shgt = _dot(hfw_ref[...], _leaky(a2)) + hfb_ref[...]
    scgt = _dot(cfw_ref[...], _leaky(c2)) + cfb_ref[...]

    h2f = _cell_t52(-dwf_ref[...] + _dot(dhf_ref[...], shgt[:_HID])
                    + dbf_ref[...], scgt[:_HID])
    h2b = _cell_t52(-dwb_ref[...] + _dot(dhb_ref[...], shgt[_HID:])
                    + dbb_ref[...], scgt[_HID:])
    headt_ref[...] = _dot(owt_ref[...], jnp.concatenate([h2f, h2b], axis=0)) \
        + obt_ref[...]


def _blk_half(w, off):
    out = jnp.zeros((4 * 32, w.shape[1]), jnp.float32)
    for j in range(4):
        out = out.at[32 * j + off:32 * j + off + _HID].set(
            w[_HID * j:_HID * (j + 1)])
    return out


def _blk_rec(wf, wb):
    out = jnp.zeros((128, 32), jnp.float32)
    for j in range(4):
        out = out.at[32 * j:32 * j + _HID, 0:_HID].set(
            wf[_HID * j:_HID * (j + 1)])
        out = out.at[32 * j + 16:32 * j + 16 + _HID, 16:16 + _HID].set(
            wb[_HID * j:_HID * (j + 1)])
    return out


def _blk_in2(wf, wb):
    ws = jnp.zeros((128, 32), jnp.float32)
    wr = jnp.zeros((128, 32), jnp.float32)
    for j in range(4):
        r = slice(_HID * j, _HID * (j + 1))
        ws = ws.at[32 * j:32 * j + _HID, 0:_HID].set(wf[r, 0:_HID])
        ws = ws.at[32 * j + 16:32 * j + 16 + _HID, 16:16 + _HID].set(
            wb[r, _HID:2 * _HID])
        wr = wr.at[32 * j:32 * j + _HID, 16:16 + _HID].set(
            wf[r, _HID:2 * _HID])
        wr = wr.at[32 * j + 16:32 * j + 16 + _HID, 0:_HID].set(wb[r, 0:_HID])
    return ws, wr


def _blk_bias(bf, bb):
    out = jnp.zeros((128, 1), jnp.float32)
    for j in range(4):
        out = out.at[32 * j:32 * j + _HID, 0].set(bf[_HID * j:_HID * (j + 1)])
        out = out.at[32 * j + 16:32 * j + 16 + _HID, 0].set(
            bb[_HID * j:_HID * (j + 1)])
    return out


def _proj_half(w):
    out = jnp.zeros((2 * _HID, 32), jnp.float32)
    out = out.at[:, 0:_HID].set(w[0:_HID].T)
    out = out.at[:, 16:16 + _HID].set(w[_HID:2 * _HID].T)
    return out


def _prep(params):
    enc = params['enc']

    def pack(l):
        lf, lb = enc[l][0], enc[l][1]
        whh = _blk_rec(lf['Whh'], lb['Whh'])
        b = _blk_bias(lf['bih'] + lf['bhh'], lb['bih'] + lb['bhh'])
        return whh, b

    whh0, b0 = pack(0)
    whh1, b1 = pack(1)
    w1s, w1r = _blk_in2(enc[1][0]['Wih'], enc[1][1]['Wih'])
    dec = params['dec']
    return dict(
        embt=params['emb'][:_VOCAB].T,
        wf0=_blk_half(enc[0][0]['Wih'], 0),
        wb0=_blk_half(enc[0][1]['Wih'], 16),
        whh0=whh0, b0=b0,
        w1f=w1s, w1b=w1r,
        whh1=whh1, b1=b1,
        p1w0=_proj_half(params['proj1_w'][:2 * _HID]),
        p1w1=_proj_half(params['proj1_w'][2 * _HID:]),
        p1b=params['proj1_b'][:, None],
        p2w0=_proj_half(params['proj2_w'][:2 * _HID]),
        p2w1=_proj_half(params['proj2_w'][2 * _HID:]),
        p2b=params['proj2_b'][:, None],
        dwih=jnp.stack([dec[k]['Wih'].T for k in ('f', 'b')]),
        db=jnp.stack([(dec[k]['bih'] + dec[k]['bhh'])[None, :]
                      for k in ('f', 'b')]),
        ow=params['out_w'], ob=params['out_b'][None, :],
        dwf=dec['f']['Wih'], dhf=dec['f']['Whh'],
        dbf=(dec['f']['bih'] + dec['f']['bhh'])[:, None],
        dwb=dec['b']['Wih'], dhb=dec['b']['Whh'],
        dbb=(dec['b']['bih'] + dec['b']['bhh'])[:, None],
        owt=params['out_w'].T, obt=params['out_b'][:, None],
    )


def _gp(g):
    return (g['w1'].T, g['b1'][:, None], g['w2'].T, g['b2'][:, None],
            g['fc_w'].T, g['fc_b'][:, None])


def kernel(params, x, edge_index):
    p = _prep(params)
    counts = _sc_counts(edge_index).reshape(2, _N, _N)

    xc = x.T
    xr = xc[:, None, :]
    xt = xc[:, :, None]
    tail, sht, sct = pl.pallas_call(
        _enc_body,
        out_shape=[
            jax.ShapeDtypeStruct((_N, _SEQ, _VOCAB), jnp.float32),
            jax.ShapeDtypeStruct((2 * _HID, _N), jnp.float32),
            jax.ShapeDtypeStruct((2 * _HID, _N), jnp.float32),
        ],
        scratch_shapes=[pltpu.VMEM((_SEQ, 2 * _HID + 6, _N), jnp.float32)],
    )(xr, xt, p['embt'], p['wf0'], p['wb0'], p['whh0'], p['b0'],
      p['w1f'], p['w1b'], p['whh1'], p['b1'],
      p['p1w0'], p['p1w1'], p['p1b'], p['p2w0'], p['p2w1'], p['p2b'],
      p['dwih'], p['db'], p['ow'], p['ob'])

    gh = _gp(params['gnn_h'])
    gc = _gp(params['gnn_c'])
    headt = pl.pallas_call(
        _gnn_dec_body,
        out_shape=jax.ShapeDtypeStruct((_VOCAB, _N), jnp.float32),
    )(counts, sht, sct, *gh[:2], *gh[2:4], *gh[4:], *gc[:2], *gc[2:4],
      *gc[4:],
      p['dwf'], p['dhf'], p['dbf'], p['dwb'], p['dhb'], p['dbb'],
      p['owt'], p['obt'])

    return tail.at[:, 0, :].set(headt.T)


# speedup vs baseline: 1.2895x; 1.2895x over previous
import functools

import jax
import jax.numpy as jnp
from jax import lax
from jax.experimental import pallas as pl
from jax.experimental.pallas import tpu as pltpu
from jax.experimental.pallas import tpu_sc as plsc

_VOCAB = 64
_IN = 64
_HID = 13
_N = 1024
_SEQ = 16
_E = 16384
_G = 4 * _HID

_HP = lax.Precision.DEFAULT


def _dot(a, b):
    return jnp.dot(a, b, precision=_HP, preferred_element_type=jnp.float32)


def _cell(g, c):
    i = jax.nn.sigmoid(g[:, 0:_HID])
    f = jax.nn.sigmoid(g[:, _HID:2 * _HID])
    gg = jnp.tanh(g[:, 2 * _HID:3 * _HID])
    o = jax.nn.sigmoid(g[:, 3 * _HID:4 * _HID])
    c2 = f * c + i * gg
    return o * jnp.tanh(c2), c2


def _cell_t(g, c):
    i = jax.nn.sigmoid(g[0:32])
    f = jax.nn.sigmoid(g[32:64])
    gg = jnp.tanh(g[64:96])
    o = jax.nn.sigmoid(g[96:128])
    c2 = f * c + i * gg
    return o * jnp.tanh(c2), c2


def _cell_t52(g, c):
    i = jax.nn.sigmoid(g[0:_HID])
    f = jax.nn.sigmoid(g[_HID:2 * _HID])
    gg = jnp.tanh(g[2 * _HID:3 * _HID])
    o = jax.nn.sigmoid(g[3 * _HID:4 * _HID])
    c2 = f * c + i * gg
    return o * jnp.tanh(c2)


def _leaky(v):
    return jnp.where(v >= 0, v, 0.01 * v)


def _sc_counts(edge_index):
    info = plsc.get_sparse_core_info()
    nc, ns = info.num_cores, info.num_subcores
    ntiles = nc * ns
    e_per = _E // ntiles
    chunk = 128
    nchunk = e_per // chunk
    words_per_tile = (_N * _N) // ns
    zb = 2048

    mesh = plsc.VectorSubcoreMesh(core_axis_name="c", subcore_axis_name="s")

    @functools.partial(
        pl.kernel,
        out_type=jax.ShapeDtypeStruct((nc, _N * _N), jnp.float32),
        mesh=mesh,
        scratch_types=[
            pltpu.VMEM((e_per,), jnp.int32),
            pltpu.VMEM((e_per,), jnp.int32),
            pltpu.VMEM((nchunk, chunk), jnp.int32),
            pltpu.VMEM((chunk,), jnp.float32),
            pltpu.VMEM((zb,), jnp.float32),
            pltpu.VMEM_SHARED((_N * _N,), jnp.float32),
        ],
    )
    def counts_kernel(edges, out, src_v, dst_v, idx_v, ones_v, zb_v, acc):
        cid = lax.axis_index("c")
        sid = lax.axis_index("s")
        gid = cid * ns + sid

        zeros16 = jnp.zeros((16,), jnp.float32)
        ones16 = jnp.ones((16,), jnp.float32)
        for k in range(zb // 16):
            zb_v[pl.ds(k * 16, 16)] = zeros16
        for k in range(chunk // 16):
            ones_v[pl.ds(k * 16, 16)] = ones16

        base = gid * e_per
        pltpu.sync_copy(edges.at[0, pl.ds(base, e_per)], src_v)
        pltpu.sync_copy(edges.at[1, pl.ds(base, e_per)], dst_v)
        for j in range(nchunk):
            for k in range(chunk // 16):
                o = j * chunk + k * 16
                s16 = src_v[pl.ds(o, 16)]
                d16 = dst_v[pl.ds(o, 16)]
                idx_v[j, pl.ds(k * 16, 16)] = s16 * _N + d16

        zbase = sid * words_per_tile
        for j in range(words_per_tile // zb):
            pltpu.sync_copy(zb_v, acc.at[pl.ds(zbase + j * zb, zb)])
        plsc.subcore_barrier()

        for j in range(nchunk):
            pltpu.sync_copy(ones_v, acc.at[idx_v.at[j]], add=True)
        plsc.subcore_barrier()

        pltpu.sync_copy(acc.at[pl.ds(zbase, words_per_tile)],
                        out.at[cid, pl.ds(zbase, words_per_tile)])

    return counts_kernel(edge_index)


def _enc_body(xr_ref, xt_ref, embt_ref, wf0_ref, wb0_ref, whh0_ref, b0_ref,
              w1f_ref, w1b_ref, whh1_ref, b1_ref,
              p1w0_ref, p1w1_ref, p1b_ref, p2w0_ref, p2w1_ref, p2b_ref,
              dwih_ref, db_ref, ow_ref, ob_ref,
              tail_ref, sht_ref, sct_ref, hs_ref):
    iota_t = lax.broadcasted_iota(jnp.int32, (_VOCAB, _N), 0)

    def oh_t(t):
        return (xr_ref[t] == iota_t).astype(jnp.float32)

    embt = embt_ref[...]
    ewf = _dot(wf0_ref[...], embt)
    ewb = _dot(wb0_ref[...], embt)
    whh0 = whh0_ref[...]
    b0 = b0_ref[...]
    z = jnp.zeros((2 * _HID + 6, _N), jnp.float32)

    def step0(s, hc):
        h, c = hc
        g = _dot(ewf, oh_t(s)) + _dot(ewb, oh_t(_SEQ - 1 - s))
        g = g + b0 + _dot(whh0, h)
        h, c = _cell_t(g, c)
        hs_ref[s] = h
        return (h, c)

    h0, c0 = lax.fori_loop(0, _SEQ, step0, (z, z))

    whh1 = whh1_ref[...]
    b1 = b1_ref[...]

    def step1(s, hc):
        h, c = hc
        g = _dot(w1f_ref[...], hs_ref[s]) + _dot(w1b_ref[...],
                                                 hs_ref[_SEQ - 1 - s])
        g = g + b1 + _dot(whh1, h)
        return _cell_t(g, c)

    h1, c1 = lax.fori_loop(0, _SEQ, step1, (z, z))

    sht_ref[...] = _dot(p1w0_ref[...], h0) + _dot(p1w1_ref[...], h1) \
        + p1b_ref[...]
    sct_ref[...] = _dot(p2w0_ref[...], c0) + _dot(p2w1_ref[...], c1) \
        + p2b_ref[...]

    vs = lax.broadcasted_iota(jnp.int32, (_VOCAB, 1), 0).astype(jnp.float32)
    zc = jnp.zeros((_VOCAB, _HID), jnp.float32)
    h2s = []
    for d in range(2):
        g = vs * dwih_ref[d] + db_ref[d]
        h2, _ = _cell(g, zc)
        h2s.append(h2)
    table = _dot(jnp.concatenate(h2s, axis=1), ow_ref[...]) + ob_ref[...]
    zt = jnp.zeros((_VOCAB, _VOCAB), jnp.float32)
    table2 = jnp.concatenate(
        [jnp.concatenate([table, zt], axis=1),
         jnp.concatenate([zt, table], axis=1)], axis=0)
    iota2 = lax.broadcasted_iota(jnp.int32, (_N, 2 * _VOCAB), 1)
    im = iota2 & (_VOCAB - 1)
    hi = iota2 >= _VOCAB
    for p in range(_SEQ // 2):
        tok = jnp.where(hi, xt_ref[2 * p], xt_ref[max(2 * p - 1, 0)])
        ohn2 = (im == tok).astype(jnp.float32)
        tail_ref[:, pl.ds(2 * _VOCAB * p, 2 * _VOCAB)] = _dot(ohn2, table2)


def _gnn_dec_body(ct_ref, sht_ref, sct_ref,
                  hw1_ref, hb1_ref, hw2_ref, hb2_ref, hfw_ref, hfb_ref,
                  cw1_ref, cb1_ref, cw2_ref, cb2_ref, cfw_ref, cfb_ref,
                  dwf_ref, dhf_ref, dbf_ref, dwb_ref, dhb_ref, dbb_ref,
                  owt_ref, obt_ref, headt_ref):
    ct = ct_ref[0] + ct_ref[1]
    deg = jnp.sum(ct, axis=0, keepdims=True) + 1.0
    dinv = lax.rsqrt(deg)
    d2 = dinv * dinv

    def conv2(hat, hct, wa_ref, ba_ref, wc_ref, bc_ref):
        wa = wa_ref[...]
        hwt = jnp.concatenate([_dot(wa, hat), _dot(wc_ref[...], hct)], axis=0)
        aggt = dinv * _dot(hwt * dinv, ct) + d2 * hwt
        k = wa.shape[0]
        return aggt[:k] + ba_ref[...], aggt[k:] + bc_ref[...]

    a1, c1 = conv2(sht_ref[...], sct_ref[...], hw1_ref, hb1_ref,
                   cw1_ref, cb1_ref)
    a2, c2 = conv2(_leaky(a1), _leaky(c1), hw2_ref, hb2_ref,
                   cw2_ref, cb2_ref)
    shgt = _dot(hfw_ref[...], _leaky(a2)) + hfb_ref[...]
    scgt = _dot(cfw_ref[...], _leaky(c2)) + cfb_ref[...]

    h2f = _cell_t52(-dwf_ref[...] + _dot(dhf_ref[...], shgt[:_HID])
                    + dbf_ref[...], scgt[:_HID])
    h2b = _cell_t52(-dwb_ref[...] + _dot(dhb_ref[...], shgt[_HID:])
                    + dbb_ref[...], scgt[_HID:])
    headt_ref[...] = _dot(owt_ref[...], jnp.concatenate([h2f, h2b], axis=0)) \
        + obt_ref[...]


def _blk_half(w, off):
    out = jnp.zeros((4 * 32, w.shape[1]), jnp.float32)
    for j in range(4):
        out = out.at[32 * j + off:32 * j + off + _HID].set(
            w[_HID * j:_HID * (j + 1)])
    return out


def _blk_rec(wf, wb):
    out = jnp.zeros((128, 32), jnp.float32)
    for j in range(4):
        out = out.at[32 * j:32 * j + _HID, 0:_HID].set(
            wf[_HID * j:_HID * (j + 1)])
        out = out.at[32 * j + 16:32 * j + 16 + _HID, 16:16 + _HID].set(
            wb[_HID * j:_HID * (j + 1)])
    return out


def _blk_in2(wf, wb):
    ws = jnp.zeros((128, 32), jnp.float32)
    wr = jnp.zeros((128, 32), jnp.float32)
    for j in range(4):
        r = slice(_HID * j, _HID * (j + 1))
        ws = ws.at[32 * j:32 * j + _HID, 0:_HID].set(wf[r, 0:_HID])
        ws = ws.at[32 * j + 16:32 * j + 16 + _HID, 16:16 + _HID].set(
            wb[r, _HID:2 * _HID])
        wr = wr.at[32 * j:32 * j + _HID, 16:16 + _HID].set(
            wf[r, _HID:2 * _HID])
        wr = wr.at[32 * j + 16:32 * j + 16 + _HID, 0:_HID].set(wb[r, 0:_HID])
    return ws, wr


def _blk_bias(bf, bb):
    out = jnp.zeros((128, 1), jnp.float32)
    for j in range(4):
        out = out.at[32 * j:32 * j + _HID, 0].set(bf[_HID * j:_HID * (j + 1)])
        out = out.at[32 * j + 16:32 * j + 16 + _HID, 0].set(
            bb[_HID * j:_HID * (j + 1)])
    return out


def _proj_half(w):
    out = jnp.zeros((2 * _HID, 32), jnp.float32)
    out = out.at[:, 0:_HID].set(w[0:_HID].T)
    out = out.at[:, 16:16 + _HID].set(w[_HID:2 * _HID].T)
    return out


def _prep(params):
    enc = params['enc']

    def pack(l):
        lf, lb = enc[l][0], enc[l][1]
        whh = _blk_rec(lf['Whh'], lb['Whh'])
        b = _blk_bias(lf['bih'] + lf['bhh'], lb['bih'] + lb['bhh'])
        return whh, b

    whh0, b0 = pack(0)
    whh1, b1 = pack(1)
    w1s, w1r = _blk_in2(enc[1][0]['Wih'], enc[1][1]['Wih'])
    dec = params['dec']
    return dict(
        embt=params['emb'][:_VOCAB].T,
        wf0=_blk_half(enc[0][0]['Wih'], 0),
        wb0=_blk_half(enc[0][1]['Wih'], 16),
        whh0=whh0, b0=b0,
        w1f=w1s, w1b=w1r,
        whh1=whh1, b1=b1,
        p1w0=_proj_half(params['proj1_w'][:2 * _HID]),
        p1w1=_proj_half(params['proj1_w'][2 * _HID:]),
        p1b=params['proj1_b'][:, None],
        p2w0=_proj_half(params['proj2_w'][:2 * _HID]),
        p2w1=_proj_half(params['proj2_w'][2 * _HID:]),
        p2b=params['proj2_b'][:, None],
        dwih=jnp.stack([dec[k]['Wih'].T for k in ('f', 'b')]),
        db=jnp.stack([(dec[k]['bih'] + dec[k]['bhh'])[None, :]
                      for k in ('f', 'b')]),
        ow=params['out_w'], ob=params['out_b'][None, :],
        dwf=dec['f']['Wih'], dhf=dec['f']['Whh'],
        dbf=(dec['f']['bih'] + dec['f']['bhh'])[:, None],
        dwb=dec['b']['Wih'], dhb=dec['b']['Whh'],
        dbb=(dec['b']['bih'] + dec['b']['bhh'])[:, None],
        owt=params['out_w'].T, obt=params['out_b'][:, None],
    )


def _gp(g):
    return (g['w1'].T, g['b1'][:, None], g['w2'].T, g['b2'][:, None],
            g['fc_w'].T, g['fc_b'][:, None])


def kernel(params, x, edge_index):
    p = _prep(params)
    counts = _sc_counts(edge_index).reshape(2, _N, _N)

    xc = x.T
    xr = xc[:, None, :]
    xt = xc[:, :, None]
    tail, sht, sct = pl.pallas_call(
        _enc_body,
        out_shape=[
            jax.ShapeDtypeStruct((_N, _SEQ * _VOCAB), jnp.float32),
            jax.ShapeDtypeStruct((2 * _HID, _N), jnp.float32),
            jax.ShapeDtypeStruct((2 * _HID, _N), jnp.float32),
        ],
        scratch_shapes=[pltpu.VMEM((_SEQ, 2 * _HID + 6, _N), jnp.float32)],
    )(xr, xt, p['embt'], p['wf0'], p['wb0'], p['whh0'], p['b0'],
      p['w1f'], p['w1b'], p['whh1'], p['b1'],
      p['p1w0'], p['p1w1'], p['p1b'], p['p2w0'], p['p2w1'], p['p2b'],
      p['dwih'], p['db'], p['ow'], p['ob'])

    gh = _gp(params['gnn_h'])
    gc = _gp(params['gnn_c'])
    headt = pl.pallas_call(
        _gnn_dec_body,
        out_shape=jax.ShapeDtypeStruct((_VOCAB, _N), jnp.float32),
    )(counts, sht, sct, *gh[:2], *gh[2:4], *gh[4:], *gc[:2], *gc[2:4],
      *gc[4:],
      p['dwf'], p['dhf'], p['dbf'], p['dwb'], p['dhb'], p['dbb'],
      p['owt'], p['obt'])

    return tail.reshape(_N, _SEQ, _VOCAB).at[:, 0, :].set(headt.T)
